# baseline (speedup 1.0000x reference)
import jax
import jax.numpy as jnp
from jax import lax
from jax.experimental import pallas as pl
from jax.experimental.pallas import tpu as pltpu
from jax.experimental.pallas import tpu_sc as plsc

D = 768
N = 4 * 4096
NC, NS = 2, 16
NW = NC * NS
PER_W = N // NW
C = 32
NCH = PER_W // C
LANES = 16
COLS = D // LANES


def _body(inp_ref, pos_ref, tok_tab, pos_tab, out_ref,
          idx_t, idx_p, tok0, tok1, pbuf0, pbuf1,
          st0, st1, sp0, sp1, sw0, sw1):
    wid = lax.axis_index("s") * NC + lax.axis_index("c")
    pltpu.sync_copy(inp_ref.at[pl.ds(wid * NCH, NCH)], idx_t)
    pltpu.sync_copy(pos_ref.at[pl.ds(wid * NCH, NCH)], idx_p)

    toks = (tok0, tok1)
    pbufs = (pbuf0, pbuf1)
    sts = (st0, st1)
    sps = (sp0, sp1)
    sws = (sw0, sw1)

    def issue(j, b):
        ct = pltpu.async_copy(tok_tab.at[idx_t.at[j]], toks[b], sts[b])
        cp = pltpu.async_copy(pos_tab.at[idx_p.at[j]], pbufs[b], sps[b])
        return ct, cp

    pending = issue(0, 0)
    wpending = [None, None]
    for j in range(NCH):
        b = j % 2
        ct, cp = pending
        ct.wait()
        cp.wait()
        if j + 1 < NCH:
            if wpending[1 - b] is not None:
                wpending[1 - b].wait()
                wpending[1 - b] = None
            pending = issue(j + 1, 1 - b)
        tb, pb = toks[b], pbufs[b]

        def add_row(r, _, tb=tb, pb=pb):
            for k in range(COLS):
                s = pl.ds(k * LANES, LANES)
                tb[r, s] = tb[r, s] + pb[r, s]
            return 0

        lax.fori_loop(0, C, add_row, 0)
        wpending[b] = pltpu.async_copy(
            tb, out_ref.at[pl.ds(wid * PER_W + j * C, C)], sws[b])
    for b in range(2):
        if wpending[b] is not None:
            wpending[b].wait()


@jax.jit
def kernel(input, pos, token_table, pos_table):
    mesh = plsc.VectorSubcoreMesh(core_axis_name="c", subcore_axis_name="s")
    k = pl.kernel(
        _body,
        mesh=mesh,
        out_type=jax.ShapeDtypeStruct((N, D), jnp.float32),
        scratch_types=[
            pltpu.VMEM((NCH, C), jnp.int32),
            pltpu.VMEM((NCH, C), jnp.int32),
            pltpu.VMEM((C, D), jnp.float32),
            pltpu.VMEM((C, D), jnp.float32),
            pltpu.VMEM((C, D), jnp.float32),
            pltpu.VMEM((C, D), jnp.float32),
            pltpu.SemaphoreType.DMA,
            pltpu.SemaphoreType.DMA,
            pltpu.SemaphoreType.DMA,
            pltpu.SemaphoreType.DMA,
            pltpu.SemaphoreType.DMA,
            pltpu.SemaphoreType.DMA,
        ],
    )
    inp2 = input.reshape(N // C, C)
    pos2 = pos.reshape(N // C, C)
    out = k(inp2, pos2, token_table, pos_table)
    return out.reshape(input.shape[0], input.shape[1], D)


# speedup vs baseline: 1.0064x; 1.0064x over previous
import jax
import jax.numpy as jnp
from jax import lax
from jax.experimental import pallas as pl
from jax.experimental.pallas import tpu as pltpu
from jax.experimental.pallas import tpu_sc as plsc

D = 768
B, S = 4, 4096
N = B * S
NC, NS = 2, 16
NW = NC * NS
PER_W = N // NW
WPB = S // PER_W
C = 32
NCH = PER_W // C
LANES = 16
COLS = D // LANES


def _body(inp_ref, pos_ref, tok_tab, pos_tab, out_ref,
          idx_t, idx_p, tok0, tok1, pbuf0, pbuf1,
          st0, st1, sp0, sp1, sw0, sw1):
    wid = lax.axis_index("s") * NC + lax.axis_index("c")
    brow = wid // WPB
    col0 = (wid % WPB) * PER_W
    pltpu.sync_copy(inp_ref.at[brow, pl.ds(col0, PER_W)], idx_t)
    pltpu.sync_copy(pos_ref.at[brow, pl.ds(col0, PER_W)], idx_p)

    toks = (tok0, tok1)
    pbufs = (pbuf0, pbuf1)
    sts = (st0, st1)
    sps = (sp0, sp1)
    sws = (sw0, sw1)

    def issue(j, b):
        ct = pltpu.async_copy(
            tok_tab.at[idx_t.at[pl.ds(j * C, C)]], toks[b], sts[b])
        cp = pltpu.async_copy(
            pos_tab.at[idx_p.at[pl.ds(j * C, C)]], pbufs[b], sps[b])
        return ct, cp

    pending = issue(0, 0)
    wpending = [None, None]
    for j in range(NCH):
        b = j % 2
        ct, cp = pending
        ct.wait()
        cp.wait()
        if j + 1 < NCH:
            if wpending[1 - b] is not None:
                wpending[1 - b].wait()
                wpending[1 - b] = None
            pending = issue(j + 1, 1 - b)
        tb, pb = toks[b], pbufs[b]

        def add_row(r, _, tb=tb, pb=pb):
            for k in range(COLS):
                s = pl.ds(k * LANES, LANES)
                tb[r, s] = tb[r, s] + pb[r, s]
            return 0

        lax.fori_loop(0, C, add_row, 0)
        wpending[b] = pltpu.async_copy(
            tb, out_ref.at[brow, pl.ds(col0 + j * C, C)], sws[b])
    for b in range(2):
        if wpending[b] is not None:
            wpending[b].wait()


@jax.jit
def kernel(input, pos, token_table, pos_table):
    mesh = plsc.VectorSubcoreMesh(core_axis_name="c", subcore_axis_name="s")
    k = pl.kernel(
        _body,
        mesh=mesh,
        out_type=jax.ShapeDtypeStruct((B, S, D), jnp.float32),
        scratch_types=[
            pltpu.VMEM((PER_W,), jnp.int32),
            pltpu.VMEM((PER_W,), jnp.int32),
            pltpu.VMEM((C, D), jnp.float32),
            pltpu.VMEM((C, D), jnp.float32),
            pltpu.VMEM((C, D), jnp.float32),
            pltpu.VMEM((C, D), jnp.float32),
            pltpu.SemaphoreType.DMA,
            pltpu.SemaphoreType.DMA,
            pltpu.SemaphoreType.DMA,
            pltpu.SemaphoreType.DMA,
            pltpu.SemaphoreType.DMA,
            pltpu.SemaphoreType.DMA,
        ],
    )
    return k(input, pos, token_table, pos_table)


# speedup vs baseline: 1.0610x; 1.0542x over previous
import jax
import jax.numpy as jnp
from jax import lax
from jax.experimental import pallas as pl
from jax.experimental.pallas import tpu as pltpu
from jax.experimental.pallas import tpu_sc as plsc

D = 768
B, S = 4, 4096
N = B * S
NC, NS = 2, 16
NW = NC * NS
PER_W = N // NW
WPB = S // PER_W
C = 16
NCH = PER_W // C
NBUF = 4
LANES = 16
COLS = D // LANES


def _body(inp_ref, pos_ref, tok_tab, pos_tab, out_ref,
          idx_t, idx_p,
          tok0, tok1, tok2, tok3, pb0, pb1, pb2, pb3,
          st0, st1, st2, st3, sp0, sp1, sp2, sp3,
          sw0, sw1, sw2, sw3):
    wid = lax.axis_index("s") * NC + lax.axis_index("c")
    brow = wid // WPB
    col0 = (wid % WPB) * PER_W
    pltpu.sync_copy(inp_ref.at[brow, pl.ds(col0, PER_W)], idx_t)
    pltpu.sync_copy(pos_ref.at[brow, pl.ds(col0, PER_W)], idx_p)

    toks = (tok0, tok1, tok2, tok3)
    pbufs = (pb0, pb1, pb2, pb3)
    sts = (st0, st1, st2, st3)
    sps = (sp0, sp1, sp2, sp3)
    sws = (sw0, sw1, sw2, sw3)

    def g_descs(j, b):
        ct = pltpu.make_async_copy(
            tok_tab.at[idx_t.at[pl.ds(j * C, C)]], toks[b], sts[b])
        cp = pltpu.make_async_copy(
            pos_tab.at[idx_p.at[pl.ds(j * C, C)]], pbufs[b], sps[b])
        return ct, cp

    def w_desc(j, b):
        return pltpu.make_async_copy(
            toks[b], out_ref.at[brow, pl.ds(col0 + j * C, C)], sws[b])

    def g_issue(j, b):
        ct, cp = g_descs(j, b)
        ct.start()
        cp.start()

    def g_wait(j, b):
        ct, cp = g_descs(j, b)
        ct.wait()
        cp.wait()

    def add(j, b):
        tb, pb = toks[b], pbufs[b]

        def add_row(r, _):
            for k in range(COLS):
                s = pl.ds(k * LANES, LANES)
                tb[r, s] = tb[r, s] + pb[r, s]
            return 0

        lax.fori_loop(0, C, add_row, 0)

    def step(j, b):
        g_wait(j, b)
        w_desc(j - 2, (b - 2) % NBUF).wait()
        g_issue(j + 2, (b + 2) % NBUF)
        add(j, b)
        w_desc(j, b).start()

    g_issue(0, 0)
    g_issue(1, 1)
    g_wait(0, 0)
    g_issue(2, 2)
    add(0, 0)
    w_desc(0, 0).start()
    g_wait(1, 1)
    g_issue(3, 3)
    add(1, 1)
    w_desc(1, 1).start()

    def mid(j2, _):
        jbase = 2 + j2 * NBUF
        for b in range(NBUF):
            step(jbase + b, (2 + b) % NBUF)
        return 0

    lax.fori_loop(0, (NCH - 4) // NBUF, mid, 0)

    for j in (NCH - 2, NCH - 1):
        b = j % NBUF
        g_wait(j, b)
        w_desc(j - 2, (b - 2) % NBUF).wait()
        add(j, b)
        w_desc(j, b).start()
    w_desc(NCH - 2, (NCH - 2) % NBUF).wait()
    w_desc(NCH - 1, (NCH - 1) % NBUF).wait()


@jax.jit
def kernel(input, pos, token_table, pos_table):
    mesh = plsc.VectorSubcoreMesh(core_axis_name="c", subcore_axis_name="s")
    k = pl.kernel(
        _body,
        mesh=mesh,
        out_type=jax.ShapeDtypeStruct((B, S, D), jnp.float32),
        scratch_types=(
            [pltpu.VMEM((PER_W,), jnp.int32)] * 2
            + [pltpu.VMEM((C, D), jnp.float32)] * (2 * NBUF)
            + [pltpu.SemaphoreType.DMA] * (3 * NBUF)
        ),
    )
    return k(input, pos, token_table, pos_table)
